# baseline (speedup 1.0000x reference)
import numpy as np

import jax
import jax.numpy as jnp
from jax.experimental import pallas as pl
from jax.experimental.pallas import tpu as pltpu

C = 96
H = 512
W = 512
R = 16
NB = H // R
NC = 2
NJ = NB // NC
BL = R * W
HL = (R + 2) * W


def _in_copies(x_hbm, xbuf, isem, g, s):
    base = g * R - 1
    return [
        pltpu.make_async_copy(
            x_hbm.at[:, jnp.clip(base + r, 0, H - 1), :],
            xbuf.at[s, :, pl.ds(r * W, W)],
            isem.at[s])
        for r in range(R + 2)
    ]


def _out_copies(out_hbm, obuf, osem, g, s):
    return [
        pltpu.make_async_copy(
            obuf.at[s, :, pl.ds(r * W, W)],
            out_hbm.at[:, g * R + r, :],
            osem.at[s])
        for r in range(R)
    ]


def _conv_block(x34, wt, tap0):
    acc = None
    for kh in range(3):
        tap = kh * 3 + tap0
        wk = wt[:, tap * C:tap * C + C]
        xk = x34[:, kh * W:kh * W + BL]
        d = jax.lax.dot_general(wk, xk, (((1,), (0,)), ((), ())),
                                preferred_element_type=jnp.float32)
        acc = d if acc is None else acc + d
    return acc


def _kernel(x_hbm, wt_ref, spa_ref, chm_ref, gn_ref, b_ref,
            out_hbm, cm_ref, xbuf, obuf, isem, osem):
    j = pl.program_id(1)
    g = pl.program_id(0) * NJ + j
    s = jax.lax.rem(j, 2)
    ns = jax.lax.rem(j + 1, 2)

    @pl.when(j == 0)
    def _():
        for c in _in_copies(x_hbm, xbuf, isem, g, s):
            c.start()

    @pl.when(j + 1 < NJ)
    def _():
        for c in _in_copies(x_hbm, xbuf, isem, g + 1, ns):
            c.start()

    for c in _in_copies(x_hbm, xbuf, isem, g, s):
        c.wait()

    @pl.when(g == 0)
    def _():
        xbuf[s, :, 0:W] = jnp.zeros((C, W), jnp.float32)

    @pl.when(g == NB - 1)
    def _():
        xbuf[s, :, (R + 1) * W:(R + 2) * W] = jnp.zeros((C, W), jnp.float32)

    t = chm_ref[...] + gn_ref[...]
    t = t - jnp.max(t, axis=1, keepdims=True)
    e = jnp.exp(t)
    cmv = e / jnp.sum(e, axis=1, keepdims=True)
    cm_ref[...] = cmv

    wt = wt_ref[...].astype(jnp.bfloat16)
    x34 = xbuf[s, :, :].astype(jnp.bfloat16)

    col = jax.lax.broadcasted_iota(jnp.int32, (1, BL), 1) % W
    m0 = (col != 0).astype(jnp.float32)
    m1 = (col != W - 1).astype(jnp.float32)

    acc = _conv_block(x34, wt, 1)
    t0 = _conv_block(x34, wt, 0)
    z = jnp.zeros((C, 1), dtype=jnp.float32)
    sr = jnp.concatenate([z, t0[:, :-1]], axis=1)
    acc = acc + sr * m0
    t2 = _conv_block(x34, wt, 2)
    sl = jnp.concatenate([t2[:, 1:], z], axis=1)
    acc = acc + sl * m1

    fea = acc + b_ref[...]
    scale = cmv[:, 0:1] * spa_ref[...] + cmv[:, 1:2]
    res = jnp.maximum(fea * scale, 0.0)

    @pl.when(j >= 2)
    def _():
        for c in _out_copies(out_hbm, obuf, osem, g - 2, s):
            c.wait()

    obuf[s, :, :] = res
    for c in _out_copies(out_hbm, obuf, osem, g, s):
        c.start()

    @pl.when(j == NJ - 1)
    def _():
        for c in _out_copies(out_hbm, obuf, osem, g - 1, ns):
            c.wait()
        for c in _out_copies(out_hbm, obuf, osem, g, s):
            c.wait()


def kernel(x0, spa_mask, Wc, b, ch_mask):
    u = jax.random.uniform(jax.random.key(1234), ch_mask.shape,
                           minval=1e-8, maxval=1.0 - 1e-8)
    gn = (-jnp.log(-jnp.log(u))).reshape(C, 2)

    x3 = x0.reshape(C, H, W)
    spa = spa_mask.reshape(1, H * W)
    wc2 = Wc.reshape(C, C * 9)
    chm = ch_mask.reshape(C, 2)
    bb = b.reshape(C, 1)

    sel = np.zeros((C * 9, C * 9), dtype=np.float32)
    for tap in range(9):
        for ci in range(C):
            sel[ci * 9 + tap, tap * C + ci] = 1.0
    wt = jnp.dot(wc2, jnp.asarray(sel))

    out, cm = pl.pallas_call(
        _kernel,
        grid=(NC, NJ),
        in_specs=[
            pl.BlockSpec(memory_space=pltpu.MemorySpace.HBM),
            pl.BlockSpec((C, C * 9), lambda p, j: (0, 0)),
            pl.BlockSpec((1, BL), lambda p, j: (0, p * NJ + j)),
            pl.BlockSpec((C, 2), lambda p, j: (0, 0)),
            pl.BlockSpec((C, 2), lambda p, j: (0, 0)),
            pl.BlockSpec((C, 1), lambda p, j: (0, 0)),
        ],
        out_specs=[
            pl.BlockSpec(memory_space=pltpu.MemorySpace.HBM),
            pl.BlockSpec((C, 2), lambda p, j: (0, 0)),
        ],
        out_shape=[
            jax.ShapeDtypeStruct((C, H, W), jnp.float32),
            jax.ShapeDtypeStruct((C, 2), jnp.float32),
        ],
        scratch_shapes=[
            pltpu.VMEM((2, C, HL), jnp.float32),
            pltpu.VMEM((2, C, BL), jnp.float32),
            pltpu.SemaphoreType.DMA((2,)),
            pltpu.SemaphoreType.DMA((2,)),
        ],
        compiler_params=pltpu.CompilerParams(
            dimension_semantics=("parallel", "arbitrary")),
    )(x3, wt, spa, chm, gn, bb)

    return (out.reshape(1, C, H, W), cm.reshape(1, C, 2))


# speedup vs baseline: 1.1091x; 1.1091x over previous
import numpy as np

import jax
import jax.numpy as jnp
from jax.experimental import pallas as pl
from jax.experimental.pallas import tpu as pltpu

C = 96
H = 512
W = 512
R = 16
NB = H // R
NC = 2
NJ = NB // NC
BL = R * W
HL = (R + 2) * W


def _in_copies(x_hbm, xbuf, isem, g, s):
    base = g * R - 1
    return [
        pltpu.make_async_copy(
            x_hbm.at[:, jnp.clip(base + r, 0, H - 1), :],
            xbuf.at[s, :, pl.ds(r * W, W)],
            isem.at[s])
        for r in range(R + 2)
    ]


def _out_copies(out_hbm, obuf, osem, g, s):
    return [
        pltpu.make_async_copy(
            obuf.at[s, :, pl.ds(r * W, W)],
            out_hbm.at[:, g * R + r, :],
            osem.at[s])
        for r in range(R)
    ]


def _conv_block(xcat, wt, kw):
    wk = wt[:, kw * 3 * C:(kw + 1) * 3 * C]
    return jax.lax.dot_general(wk, xcat, (((1,), (0,)), ((), ())),
                               preferred_element_type=jnp.float32)


def _kernel(x_hbm, wt_ref, spa_ref, chm_ref, gn_ref, b_ref,
            out_hbm, cm_ref, xbuf, obuf, isem, osem):
    j = pl.program_id(1)
    g = pl.program_id(0) * NJ + j
    s = jax.lax.rem(j, 2)
    ns = jax.lax.rem(j + 1, 2)

    @pl.when(j == 0)
    def _():
        for c in _in_copies(x_hbm, xbuf, isem, g, s):
            c.start()

    @pl.when(j + 1 < NJ)
    def _():
        for c in _in_copies(x_hbm, xbuf, isem, g + 1, ns):
            c.start()

    for c in _in_copies(x_hbm, xbuf, isem, g, s):
        c.wait()

    @pl.when(g == 0)
    def _():
        xbuf[s, :, 0:W] = jnp.zeros((C, W), jnp.float32)

    @pl.when(g == NB - 1)
    def _():
        xbuf[s, :, (R + 1) * W:(R + 2) * W] = jnp.zeros((C, W), jnp.float32)

    t = chm_ref[...] + gn_ref[...]
    t = t - jnp.max(t, axis=1, keepdims=True)
    e = jnp.exp(t)
    cmv = e / jnp.sum(e, axis=1, keepdims=True)
    cm_ref[...] = cmv

    wt = wt_ref[...].astype(jnp.bfloat16)
    x34 = xbuf[s, :, :].astype(jnp.bfloat16)
    xcat = jnp.concatenate(
        [x34[:, kh * W:kh * W + BL] for kh in range(3)], axis=0)

    col = jax.lax.broadcasted_iota(jnp.int32, (1, BL), 1) % W
    m0 = (col != 0).astype(jnp.float32)
    m1 = (col != W - 1).astype(jnp.float32)

    acc = _conv_block(xcat, wt, 1)
    t0 = _conv_block(xcat, wt, 0)
    z = jnp.zeros((C, 1), dtype=jnp.float32)
    sr = jnp.concatenate([z, t0[:, :-1]], axis=1)
    acc = acc + sr * m0
    t2 = _conv_block(xcat, wt, 2)
    sl = jnp.concatenate([t2[:, 1:], z], axis=1)
    acc = acc + sl * m1

    fea = acc + b_ref[...]
    scale = cmv[:, 0:1] * spa_ref[...] + cmv[:, 1:2]
    res = jnp.maximum(fea * scale, 0.0)

    @pl.when(j >= 2)
    def _():
        for c in _out_copies(out_hbm, obuf, osem, g - 2, s):
            c.wait()

    obuf[s, :, :] = res
    for c in _out_copies(out_hbm, obuf, osem, g, s):
        c.start()

    @pl.when(j == NJ - 1)
    def _():
        for c in _out_copies(out_hbm, obuf, osem, g - 1, ns):
            c.wait()
        for c in _out_copies(out_hbm, obuf, osem, g, s):
            c.wait()


def kernel(x0, spa_mask, Wc, b, ch_mask):
    u = jax.random.uniform(jax.random.key(1234), ch_mask.shape,
                           minval=1e-8, maxval=1.0 - 1e-8)
    gn = (-jnp.log(-jnp.log(u))).reshape(C, 2)

    x3 = x0.reshape(C, H, W)
    spa = spa_mask.reshape(1, H * W)
    wc2 = Wc.reshape(C, C * 9)
    chm = ch_mask.reshape(C, 2)
    bb = b.reshape(C, 1)

    sel = np.zeros((C * 9, C * 9), dtype=np.float32)
    for kh in range(3):
        for kw in range(3):
            for ci in range(C):
                sel[ci * 9 + kh * 3 + kw, kw * 3 * C + kh * C + ci] = 1.0
    wt = jnp.dot(wc2, jnp.asarray(sel))

    out, cm = pl.pallas_call(
        _kernel,
        grid=(NC, NJ),
        in_specs=[
            pl.BlockSpec(memory_space=pltpu.MemorySpace.HBM),
            pl.BlockSpec((C, C * 9), lambda p, j: (0, 0)),
            pl.BlockSpec((1, BL), lambda p, j: (0, p * NJ + j)),
            pl.BlockSpec((C, 2), lambda p, j: (0, 0)),
            pl.BlockSpec((C, 2), lambda p, j: (0, 0)),
            pl.BlockSpec((C, 1), lambda p, j: (0, 0)),
        ],
        out_specs=[
            pl.BlockSpec(memory_space=pltpu.MemorySpace.HBM),
            pl.BlockSpec((C, 2), lambda p, j: (0, 0)),
        ],
        out_shape=[
            jax.ShapeDtypeStruct((C, H, W), jnp.float32),
            jax.ShapeDtypeStruct((C, 2), jnp.float32),
        ],
        scratch_shapes=[
            pltpu.VMEM((2, C, HL), jnp.float32),
            pltpu.VMEM((2, C, BL), jnp.float32),
            pltpu.SemaphoreType.DMA((2,)),
            pltpu.SemaphoreType.DMA((2,)),
        ],
        compiler_params=pltpu.CompilerParams(
            dimension_semantics=("parallel", "arbitrary")),
    )(x3, wt, spa, chm, gn, bb)

    return (out.reshape(1, C, H, W), cm.reshape(1, C, 2))


# speedup vs baseline: 1.1096x; 1.0004x over previous
import numpy as np

import jax
import jax.numpy as jnp
from jax.experimental import pallas as pl
from jax.experimental.pallas import tpu as pltpu

C = 96
H = 512
W = 512
R = 16
NB = H // R
NC = 2
NJ = NB // NC
BL = R * W
HL = (R + 2) * W


def _in_copies(x_hbm, xbuf, isem, g, s):
    base = g * R - 1
    return [
        pltpu.make_async_copy(
            x_hbm.at[:, jnp.clip(base + r, 0, H - 1), :],
            xbuf.at[s, :, pl.ds(r * W, W)],
            isem.at[s])
        for r in range(R + 2)
    ]


def _out_copies(out_hbm, obuf, osem, g, s):
    return [
        pltpu.make_async_copy(
            obuf.at[s, :, pl.ds(r * W, W)],
            out_hbm.at[:, g * R + r, :],
            osem.at[s])
        for r in range(R)
    ]


def _conv_block(xcat, wt, kw):
    wk = wt[:, kw * 3 * C:(kw + 1) * 3 * C]
    return jax.lax.dot_general(wk, xcat, (((1,), (0,)), ((), ())),
                               preferred_element_type=jnp.float32)


def _kernel(x_hbm, wt_ref, spa_ref, chm_ref, gn_ref, b_ref,
            out_hbm, cm_ref, xbuf, obuf, isem, osem):
    j = pl.program_id(1)
    g = pl.program_id(0) * NJ + j
    s = jax.lax.rem(j, 3)
    so = jax.lax.rem(j, 2)
    ns = jax.lax.rem(j + 1, 2)

    @pl.when(j == 0)
    def _():
        for c in _in_copies(x_hbm, xbuf, isem, g, s):
            c.start()
        for c in _in_copies(x_hbm, xbuf, isem, g + 1, jax.lax.rem(j + 1, 3)):
            c.start()

    @pl.when(j + 2 < NJ)
    def _():
        for c in _in_copies(x_hbm, xbuf, isem, g + 2, jax.lax.rem(j + 2, 3)):
            c.start()

    for c in _in_copies(x_hbm, xbuf, isem, g, s):
        c.wait()

    @pl.when(g == 0)
    def _():
        xbuf[s, :, 0:W] = jnp.zeros((C, W), jnp.float32)

    @pl.when(g == NB - 1)
    def _():
        xbuf[s, :, (R + 1) * W:(R + 2) * W] = jnp.zeros((C, W), jnp.float32)

    t = chm_ref[...] + gn_ref[...]
    t = t - jnp.max(t, axis=1, keepdims=True)
    e = jnp.exp(t)
    cmv = e / jnp.sum(e, axis=1, keepdims=True)
    cm_ref[...] = cmv

    wt = wt_ref[...].astype(jnp.bfloat16)
    x34 = xbuf[s, :, :].astype(jnp.bfloat16)
    xcat = jnp.concatenate(
        [x34[:, kh * W:kh * W + BL] for kh in range(3)], axis=0)

    col = jax.lax.broadcasted_iota(jnp.int32, (1, BL), 1) % W
    m0 = (col != 0).astype(jnp.float32)
    m1 = (col != W - 1).astype(jnp.float32)

    acc = _conv_block(xcat, wt, 1)
    t0 = _conv_block(xcat, wt, 0)
    z = jnp.zeros((C, 1), dtype=jnp.float32)
    sr = jnp.concatenate([z, t0[:, :-1]], axis=1)
    acc = acc + sr * m0
    t2 = _conv_block(xcat, wt, 2)
    sl = jnp.concatenate([t2[:, 1:], z], axis=1)
    acc = acc + sl * m1

    fea = acc + b_ref[...]
    scale = cmv[:, 0:1] * spa_ref[...] + cmv[:, 1:2]
    res = jnp.maximum(fea * scale, 0.0)

    @pl.when(j >= 2)
    def _():
        for c in _out_copies(out_hbm, obuf, osem, g - 2, so):
            c.wait()

    obuf[so, :, :] = res
    for c in _out_copies(out_hbm, obuf, osem, g, so):
        c.start()

    @pl.when(j == NJ - 1)
    def _():
        for c in _out_copies(out_hbm, obuf, osem, g - 1, ns):
            c.wait()
        for c in _out_copies(out_hbm, obuf, osem, g, so):
            c.wait()


def kernel(x0, spa_mask, Wc, b, ch_mask):
    u = jax.random.uniform(jax.random.key(1234), ch_mask.shape,
                           minval=1e-8, maxval=1.0 - 1e-8)
    gn = (-jnp.log(-jnp.log(u))).reshape(C, 2)

    x3 = x0.reshape(C, H, W)
    spa = spa_mask.reshape(1, H * W)
    wc2 = Wc.reshape(C, C * 9)
    chm = ch_mask.reshape(C, 2)
    bb = b.reshape(C, 1)

    sel = np.zeros((C * 9, C * 9), dtype=np.float32)
    for kh in range(3):
        for kw in range(3):
            for ci in range(C):
                sel[ci * 9 + kh * 3 + kw, kw * 3 * C + kh * C + ci] = 1.0
    wt = jnp.dot(wc2, jnp.asarray(sel))

    out, cm = pl.pallas_call(
        _kernel,
        grid=(NC, NJ),
        in_specs=[
            pl.BlockSpec(memory_space=pltpu.MemorySpace.HBM),
            pl.BlockSpec((C, C * 9), lambda p, j: (0, 0)),
            pl.BlockSpec((1, BL), lambda p, j: (0, p * NJ + j)),
            pl.BlockSpec((C, 2), lambda p, j: (0, 0)),
            pl.BlockSpec((C, 2), lambda p, j: (0, 0)),
            pl.BlockSpec((C, 1), lambda p, j: (0, 0)),
        ],
        out_specs=[
            pl.BlockSpec(memory_space=pltpu.MemorySpace.HBM),
            pl.BlockSpec((C, 2), lambda p, j: (0, 0)),
        ],
        out_shape=[
            jax.ShapeDtypeStruct((C, H, W), jnp.float32),
            jax.ShapeDtypeStruct((C, 2), jnp.float32),
        ],
        scratch_shapes=[
            pltpu.VMEM((3, C, HL), jnp.float32),
            pltpu.VMEM((2, C, BL), jnp.float32),
            pltpu.SemaphoreType.DMA((3,)),
            pltpu.SemaphoreType.DMA((2,)),
        ],
        compiler_params=pltpu.CompilerParams(
            dimension_semantics=("parallel", "arbitrary")),
    )(x3, wt, spa, chm, gn, bb)

    return (out.reshape(1, C, H, W), cm.reshape(1, C, 2))
